# Initial kernel scaffold; baseline (speedup 1.0000x reference)
import jax
import jax.numpy as jnp
from jax.experimental import pallas as pl


def kernel(x, edge_index, params):
    raise NotImplementedError("write your pallas kernel here")


# speedup vs baseline: 4.4456x; 4.4456x over previous
import functools
import jax
import jax.numpy as jnp
import numpy as np
from jax import lax
from jax.experimental import pallas as pl
from jax.experimental.pallas import tpu as pltpu

_PH = lax.Precision.HIGHEST


def _mm(a, b):
    return jnp.matmul(a, b, precision=_PH)


_N = 10000
_E = 320000
_INDIM = 128
_OUT = 256
_H = 5
_D8 = _OUT // 8
_D4 = _OUT // 4
_D2 = _OUT // 2
_D78 = 7 * _OUT // 8
_CAT = _OUT + _D4

_RB = 400
_GRID = _N // _RB


def _row_spec(c):
    return pl.BlockSpec((_RB, c), lambda i: (i, 0))


def _full_spec(shape):
    return pl.BlockSpec(shape, lambda i: tuple(0 for _ in shape))


def _leaky(x, s):
    return jnp.where(x > 0, x, s * x)


def _t1_body(x_ref, W1, b1, W2, b2, W3, b3, h_ref, st_ref):
    x = x_ref[...]
    h = jax.nn.softplus(_mm(x, W1[...]) + b1[...])
    h = jax.nn.softplus(_mm(h, W2[...]) + b2[...])
    h = _mm(h, W3[...]) + b3[...]
    h_ref[...] = h

    @pl.when(pl.program_id(0) == 0)
    def _():
        st_ref[...] = jnp.zeros_like(st_ref)

    st = jnp.concatenate(
        [jnp.sum(h, 0, keepdims=True), jnp.sum(h * h, 0, keepdims=True),
         jnp.zeros((6, h.shape[1]), jnp.float32)], axis=0)
    st_ref[...] += st


def _t1(x, p):
    return pl.pallas_call(
        _t1_body,
        grid=(_GRID,),
        in_specs=[_row_spec(_INDIM), _full_spec((_INDIM, 2 * _INDIM)),
                  _full_spec((1, 2 * _INDIM)), _full_spec((2 * _INDIM, _D4)),
                  _full_spec((1, _D4)), _full_spec((_D4, _D8)),
                  _full_spec((1, _D8))],
        out_specs=[_row_spec(_D8), _full_spec((8, _D8))],
        out_shape=[jax.ShapeDtypeStruct((_N, _D8), jnp.float32),
                   jax.ShapeDtypeStruct((8, _D8), jnp.float32)],
    )(x, p['nn1_W1'], p['nn1_b1'][None], p['nn1_W2'], p['nn1_b2'][None],
      p['nn1_W3'], p['nn1_b3'][None])


def _t2_body(h_ref, st_ref, deg_ref, g_ref, b_ref, W_ref,
             x0_ref, hwd_ref, dinv_ref):
    st = st_ref[...]
    mean = st[0:1, :] / _N
    var = st[1:2, :] / _N - mean * mean
    h = (h_ref[...] - mean) / jnp.sqrt(var + 1e-5) * g_ref[...] + b_ref[...]
    x0 = _leaky(h, 0.01)
    x0_ref[...] = x0
    deg = deg_ref[...]
    dinv = jnp.where(deg > 0, lax.rsqrt(deg), 0.0)
    dinv_ref[...] = dinv
    hwd_ref[...] = _mm(x0, W_ref[...]) * dinv


def _t2(h, st, deg, p):
    return pl.pallas_call(
        _t2_body,
        grid=(_GRID,),
        in_specs=[_row_spec(_D8), _full_spec((8, _D8)), _row_spec(1),
                  _full_spec((1, _D8)), _full_spec((1, _D8)),
                  _full_spec((_D8, _D4))],
        out_specs=[_row_spec(_D8), _row_spec(_D4), _row_spec(1)],
        out_shape=[jax.ShapeDtypeStruct((_N, _D8), jnp.float32),
                   jax.ShapeDtypeStruct((_N, _D4), jnp.float32),
                   jax.ShapeDtypeStruct((_N, 1), jnp.float32)],
    )(h, st, deg, p['bn0_g'][None], p['bn0_b'][None], p['gcn1_W'])


def _t3_body(agg_ref, dinv_ref, b_ref, W_ref, as_ref, ad_ref,
             hw_ref, als_ref, ald_ref):
    x1g = agg_ref[...] * dinv_ref[...] + b_ref[...]
    W = W_ref[...]
    hw = _mm(x1g, W)
    hw_ref[...] = hw
    Wr = W.reshape(W.shape[0], _H, -1)
    As = jnp.einsum('dhc,hc->dh', Wr, as_ref[...], precision=_PH)
    Ad = jnp.einsum('dhc,hc->dh', Wr, ad_ref[...], precision=_PH)
    als_ref[...] = _mm(x1g, As)
    ald_ref[...] = _mm(x1g, Ad)


def _t3(agg, dinv, b, W, a_s, a_d, din, cout):
    return pl.pallas_call(
        _t3_body,
        grid=(_GRID,),
        in_specs=[_row_spec(din), _row_spec(1), _full_spec((1, din)),
                  _full_spec((din, _H * cout)), _full_spec((_H, cout)),
                  _full_spec((_H, cout))],
        out_specs=[_row_spec(_H * cout), _row_spec(_H), _row_spec(_H)],
        out_shape=[jax.ShapeDtypeStruct((_N, _H * cout), jnp.float32),
                   jax.ShapeDtypeStruct((_N, _H), jnp.float32),
                   jax.ShapeDtypeStruct((_N, _H), jnp.float32)],
    )(agg, dinv, b, W, a_s, a_d)


def _t4a_body(m_ref, b_ref, x1p_ref, st_ref):
    x1p = m_ref[...] * (1.0 / _H) + b_ref[...]
    x1p_ref[...] = x1p

    @pl.when(pl.program_id(0) == 0)
    def _():
        st_ref[...] = jnp.zeros_like(st_ref)

    st = jnp.concatenate(
        [jnp.sum(x1p, 0, keepdims=True), jnp.sum(x1p * x1p, 0, keepdims=True),
         jnp.zeros((6, x1p.shape[1]), jnp.float32)], axis=0)
    st_ref[...] += st


def _t4a(m, b, c):
    return pl.pallas_call(
        _t4a_body,
        grid=(_GRID,),
        in_specs=[_row_spec(c), _full_spec((1, c))],
        out_specs=[_row_spec(c), _full_spec((8, c))],
        out_shape=[jax.ShapeDtypeStruct((_N, c), jnp.float32),
                   jax.ShapeDtypeStruct((8, c), jnp.float32)],
    )(m, b)


def _t4b_body(x1p_ref, st_ref, g_ref, b_ref, x0_ref, Wa_ref, Wb_ref,
              dinv_ref, x1_ref, hwd_ref):
    st = st_ref[...]
    mean = st[0:1, :] / _N
    var = st[1:2, :] / _N - mean * mean
    h = (x1p_ref[...] - mean) / jnp.sqrt(var + 1e-5) * g_ref[...] + b_ref[...]
    x1 = _leaky(h, 0.01)
    x1_ref[...] = x1
    hwd_ref[...] = (_mm(x0_ref[...], Wa_ref[...]) + _mm(x1, Wb_ref[...])) * dinv_ref[...]


def _t4b(x1p, st, x0, dinv, p):
    W = p['gcn2_W']
    return pl.pallas_call(
        _t4b_body,
        grid=(_GRID,),
        in_specs=[_row_spec(_D4), _full_spec((8, _D4)), _full_spec((1, _D4)),
                  _full_spec((1, _D4)), _row_spec(_D8),
                  _full_spec((_D8, _D2)), _full_spec((_D4, _D2)),
                  _row_spec(1)],
        out_specs=[_row_spec(_D4), _row_spec(_D2)],
        out_shape=[jax.ShapeDtypeStruct((_N, _D4), jnp.float32),
                   jax.ShapeDtypeStruct((_N, _D2), jnp.float32)],
    )(x1p, st, p['bn1_g'][None], p['bn1_b'][None], x0,
      W[:_D8], W[_D8:], dinv)


def _t6_body(x2p_ref, x0_ref, x1_ref, W1a, W1b, W1c, b1, W2, b2, W3, b3,
             W4, b4, x2_ref, pr_ref):
    x2 = _leaky(x2p_ref[...], 0.01)
    x2_ref[...] = x2
    h = (_mm(x0_ref[...], W1a[...]) + _mm(x1_ref[...], W1b[...]) + _mm(x2, W1c[...])
         + b1[...])
    h = jax.nn.softplus(h)
    h = jax.nn.softplus(_mm(h, W2[...]) + b2[...])
    h = jax.nn.softplus(_mm(h, W3[...]) + b3[...])
    pr_ref[...] = jax.nn.sigmoid(_mm(h, W4[...]) + b4[...])


def _t6(x2p, x0, x1, p):
    W1 = p['pr_W1']
    c2, c4, c8 = _CAT // 2, _CAT // 4, _CAT // 8
    return pl.pallas_call(
        _t6_body,
        grid=(_GRID,),
        in_specs=[_row_spec(_D78), _row_spec(_D8), _row_spec(_D4),
                  _full_spec((_D8, c2)), _full_spec((_D4, c2)),
                  _full_spec((_D78, c2)), _full_spec((1, c2)),
                  _full_spec((c2, c4)), _full_spec((1, c4)),
                  _full_spec((c4, c8)), _full_spec((1, c8)),
                  _full_spec((c8, 1)), _full_spec((1, 1))],
        out_specs=[_row_spec(_D78), _row_spec(1)],
        out_shape=[jax.ShapeDtypeStruct((_N, _D78), jnp.float32),
                   jax.ShapeDtypeStruct((_N, 1), jnp.float32)],
    )(x2p, x0, x1, W1[:_D8], W1[_D8:_D8 + _D4], W1[_D8 + _D4:],
      p['pr_b1'][None], p['pr_W2'], p['pr_b2'][None], p['pr_W3'],
      p['pr_b3'][None], p['pr_W4'], p['pr_b4'][None])


def _seg_sum(vals, dst):
    return jax.ops.segment_sum(vals, dst, num_segments=_N)


def _gat_msg(hw, als, ald, src, dst, c):
    alpha = _leaky(als[src] + ald[dst], 0.2)
    ex = jnp.exp(alpha)
    den = _seg_sum(ex, dst)
    coef = ex / (den[dst] + 1e-16)
    hwr = hw.reshape(_N, _H, c)
    msg = jnp.einsum('eh,ehc->ec', coef, hwr[src], precision=_PH)
    return _seg_sum(msg, dst)


def kernel(x, edge_index, params):
    p = params
    loop = jnp.arange(_N, dtype=edge_index.dtype)
    src = jnp.concatenate([edge_index[0], loop])
    dst = jnp.concatenate([edge_index[1], loop])

    h, st0 = _t1(x, p)
    deg = _seg_sum(jnp.ones_like(dst, jnp.float32), dst)[:, None]
    x0, hwd1, dinv = _t2(h, st0, deg, p)

    agg1 = _seg_sum(hwd1[src], dst)
    hw1, als1, ald1 = _t3(agg1, dinv, p['gcn1_b'][None], p['gat1_W'],
                          p['gat1_as'], p['gat1_ad'], _D4, _D4)
    m1 = _gat_msg(hw1, als1, ald1, src, dst, _D4)
    x1p, st1 = _t4a(m1, p['gat1_b'][None], _D4)
    x1, hwd2 = _t4b(x1p, st1, x0, dinv, p)

    agg2 = _seg_sum(hwd2[src], dst)
    hw2, als2, ald2 = _t3(agg2, dinv, p['gcn2_b'][None], p['gat2_W'],
                          p['gat2_as'], p['gat2_ad'], _D2, _D78)
    m2 = _gat_msg(hw2, als2, ald2, src, dst, _D78)
    x2p, _ = _t4a(m2, p['gat2_b'][None], _D78)

    x2, probs = _t6(x2p, x0, x1, p)
    xf = jnp.concatenate([x0, x1, x2], axis=1)
    return xf, probs


# speedup vs baseline: 8.2506x; 1.8559x over previous
import functools
import jax
import jax.numpy as jnp
import numpy as np
from jax import lax
from jax.experimental import pallas as pl
from jax.experimental.pallas import tpu as pltpu
from jax.experimental.pallas import tpu_sc as plsc

_N = 10000
_E = 320000
_INDIM = 128
_OUT = 256
_H = 5
_D8 = _OUT // 8
_D4 = _OUT // 4
_D2 = _OUT // 2
_D78 = 7 * _OUT // 8
_CAT = _OUT + _D4

_RB = 400
_GRID = _N // _RB

_NPAD = 10112
_EP = 331776
_K2 = 128
_K4 = 32
_RPT = _NPAD // 16

_scparams = pltpu.CompilerParams(needs_layout_passes=False,
                                use_tc_tiling_on_sc=False)


@functools.lru_cache(maxsize=None)
def _scmesh():
    return plsc.VectorSubcoreMesh(core_axis_name="c", subcore_axis_name="s")


def _mm(a, b):
    return jnp.matmul(a, b)


def _row_spec(c):
    return pl.BlockSpec((_RB, c), lambda i: (i, 0))


def _full_spec(shape):
    return pl.BlockSpec(shape, lambda i: tuple(0 for _ in shape))


def _leaky(x, s):
    return jnp.where(x > 0, x, s * x)


def _zero16(ref, n):
    def body(i, carry):
        ref[pl.ds(i * 16, 16)] = jnp.zeros((16,), jnp.float32)
        return carry
    lax.fori_loop(0, n // 16, body, 0)


def _zero2d(ref, rows, cols):
    def body(i, carry):
        for ch in range(cols // 16):
            ref[i, pl.ds(ch * 16, 16)] = jnp.zeros((16,), jnp.float32)
        return carry
    lax.fori_loop(0, rows, body, 0)


def _bc(v, lane):
    idx = jnp.full((16, 1), lane, jnp.int32)
    dn = lax.GatherDimensionNumbers(offset_dims=(), collapsed_slice_dims=(0,),
                                    start_index_map=(0,))
    return lax.gather(v, idx, dn, (1,),
                      mode=lax.GatherScatterMode.PROMISE_IN_BOUNDS)


@functools.lru_cache(maxsize=None)
def _make_deg():
    @functools.partial(
        pl.kernel, mesh=_scmesh(),
        out_type=jax.ShapeDtypeStruct((32, _NPAD), jnp.float32),
        scratch_types=[pltpu.VMEM((_NPAD,), jnp.float32),
                       pltpu.VMEM((_K2,), jnp.int32)],
        compiler_params=_scparams)
    def k(dst_hbm, out_hbm, tab_v, idx_v):
        c = lax.axis_index("c")
        s = lax.axis_index("s")
        w = s * 2 + c
        _zero16(tab_v, _NPAD)
        ones = jnp.full((16,), 1.0, jnp.float32)
        npt = _EP // 32
        base0 = w * npt

        def body(wi, carry):
            pltpu.sync_copy(dst_hbm.at[pl.ds(base0 + wi * _K2, _K2)], idx_v)
            for b in range(_K2 // 16):
                plsc.addupdate_scatter(tab_v, [idx_v[pl.ds(b * 16, 16)]],
                                       ones)
            return carry
        lax.fori_loop(0, npt // _K2, body, 0)
        pltpu.sync_copy(tab_v, out_hbm.at[w])
    return k


@functools.lru_cache(maxsize=None)
def _make_gcn(C):
    @functools.partial(
        pl.kernel, mesh=_scmesh(),
        out_type=jax.ShapeDtypeStruct((2, _NPAD, C), jnp.float32),
        scratch_types=[pltpu.VMEM((_K2,), jnp.int32),
                       pltpu.VMEM((_K2,), jnp.int32),
                       pltpu.VMEM((_K2, C), jnp.float32),
                       pltpu.VMEM((8, C), jnp.float32),
                       pltpu.VMEM_SHARED((_NPAD, C), jnp.float32),
                       pltpu.SemaphoreType.DMA],
        compiler_params=_scparams)
    def k(tab_hbm, src_hbm, dst_hbm, out_hbm,
          sidx_v, didx_v, rows_v, zbuf_v, acc_sh, sem):
        c = lax.axis_index("c")
        s = lax.axis_index("s")
        w = s * 2 + c
        _zero2d(zbuf_v, 8, C)

        def zbody(i, carry):
            pltpu.sync_copy(zbuf_v, acc_sh.at[pl.ds(s * _RPT + i * 8, 8)])
            return carry
        lax.fori_loop(0, _RPT // 8, zbody, 0)
        plsc.subcore_barrier()

        npt = _EP // 32
        base0 = w * npt

        def body(wi, carry):
            base = base0 + wi * _K2
            pltpu.sync_copy(src_hbm.at[pl.ds(base, _K2)], sidx_v)
            pltpu.async_copy(tab_hbm.at[sidx_v], rows_v, sem).wait()
            pltpu.sync_copy(dst_hbm.at[pl.ds(base, _K2)], didx_v)
            pltpu.sync_copy(rows_v, acc_sh.at[didx_v], add=True)
            return carry
        lax.fori_loop(0, npt // _K2, body, 0)
        plsc.subcore_barrier()
        pltpu.sync_copy(acc_sh.at[pl.ds(s * _RPT, _RPT)],
                        out_hbm.at[c, pl.ds(s * _RPT, _RPT)])
    return k


@functools.lru_cache(maxsize=None)
def _make_den():
    @functools.partial(
        pl.kernel, mesh=_scmesh(),
        out_type=jax.ShapeDtypeStruct((16, 2, 4, _NPAD), jnp.float32),
        scratch_types=[pltpu.VMEM((4, _NPAD), jnp.float32),
                       pltpu.VMEM((64,), jnp.int32),
                       pltpu.VMEM((64,), jnp.int32),
                       pltpu.VMEM((64, 16), jnp.float32),
                       pltpu.VMEM((64, 16), jnp.float32),
                       pltpu.SemaphoreType.DMA,
                       pltpu.SemaphoreType.DMA],
        compiler_params=_scparams)
    def k(alsa_hbm, alsb_hbm, alda_hbm, aldb_hbm, src_hbm, dst_hbm, out_hbm,
          den_v, sidx_v, didx_v, alsr_v, aldr_v, sem, sem2):
        c = lax.axis_index("c")
        s = lax.axis_index("s")
        for j in range(4):
            _zero16(den_v.at[j], _NPAD)

        npt = _EP // 16
        base0 = s * npt
        iota = lax.iota(jnp.int32, 16)

        def body(wi, carry):
            base = base0 + wi * 64
            pltpu.sync_copy(src_hbm.at[pl.ds(base, 64)], sidx_v)
            pltpu.sync_copy(dst_hbm.at[pl.ds(base, 64)], didx_v)

            @pl.when(c == 0)
            def _():
                pltpu.async_copy(alsa_hbm.at[sidx_v], alsr_v, sem).wait()
                pltpu.async_copy(alda_hbm.at[didx_v], aldr_v, sem2).wait()

            @pl.when(c == 1)
            def _():
                pltpu.async_copy(alsb_hbm.at[sidx_v], alsr_v, sem).wait()
                pltpu.async_copy(aldb_hbm.at[didx_v], aldr_v, sem2).wait()

            for b in range(4):
                lanes = iota + b * 16
                didx = didx_v[pl.ds(b * 16, 16)]
                for j in range(3):
                    jf = jnp.full((16,), j, jnp.int32)
                    a = (plsc.load_gather(alsr_v, [lanes, jf])
                         + plsc.load_gather(aldr_v, [lanes, jf]))
                    a = jnp.where(a > 0, a, 0.2 * a)
                    plsc.addupdate_scatter(den_v, [jf, didx], jnp.exp(a))
            return carry
        lax.fori_loop(0, npt // 64, body, 0)
        pltpu.sync_copy(den_v, out_hbm.at[s, c])
    return k


@functools.lru_cache(maxsize=None)
def _make_gat(C):
    @functools.partial(
        pl.kernel, mesh=_scmesh(),
        out_type=jax.ShapeDtypeStruct((2, _NPAD, C), jnp.float32),
        scratch_types=[pltpu.VMEM((_K4,), jnp.int32),
                       pltpu.VMEM((_K4,), jnp.int32),
                       pltpu.VMEM((_K4, 16), jnp.float32),
                       pltpu.VMEM((_K4, 16), jnp.float32),
                       pltpu.VMEM((_K4, 16), jnp.float32),
                       pltpu.VMEM((_K4, 3, C), jnp.float32),
                       pltpu.VMEM((_K4, C), jnp.float32),
                       pltpu.VMEM((8, C), jnp.float32),
                       pltpu.VMEM_SHARED((_NPAD, C), jnp.float32),
                       pltpu.SemaphoreType.DMA,
                       pltpu.SemaphoreType.DMA,
                       pltpu.SemaphoreType.DMA,
                       pltpu.SemaphoreType.DMA],
        compiler_params=_scparams)
    def k(hwa_hbm, hwb_hbm, alsa_hbm, alsb_hbm, alda_hbm, aldb_hbm,
          rina_hbm, rinb_hbm, src_hbm, dst_hbm, out_hbm,
          sidx_v, didx_v, alsr_v, aldr_v, rinr_v, rows_v, msg_v, zbuf_v,
          acc_sh, sem, sem2, sem3, sem4):
        c = lax.axis_index("c")
        s = lax.axis_index("s")
        _zero2d(zbuf_v, 8, C)

        def zbody(i, carry):
            pltpu.sync_copy(zbuf_v, acc_sh.at[pl.ds(s * _RPT + i * 8, 8)])
            return carry
        lax.fori_loop(0, _RPT // 8, zbody, 0)
        plsc.subcore_barrier()

        npt = _EP // 16
        base0 = s * npt
        iota = lax.iota(jnp.int32, 16)

        def body(wi, carry):
            base = base0 + wi * _K4
            pltpu.sync_copy(src_hbm.at[pl.ds(base, _K4)], sidx_v)
            pltpu.sync_copy(dst_hbm.at[pl.ds(base, _K4)], didx_v)

            @pl.when(c == 0)
            def _():
                pltpu.async_copy(hwa_hbm.at[sidx_v], rows_v, sem).wait()
                pltpu.async_copy(alsa_hbm.at[sidx_v], alsr_v, sem2).wait()
                pltpu.async_copy(alda_hbm.at[didx_v], aldr_v, sem3).wait()
                pltpu.async_copy(rina_hbm.at[didx_v], rinr_v, sem4).wait()

            @pl.when(c == 1)
            def _():
                pltpu.async_copy(hwb_hbm.at[sidx_v], rows_v, sem).wait()
                pltpu.async_copy(alsb_hbm.at[sidx_v], alsr_v, sem2).wait()
                pltpu.async_copy(aldb_hbm.at[didx_v], aldr_v, sem3).wait()
                pltpu.async_copy(rinb_hbm.at[didx_v], rinr_v, sem4).wait()

            coef = []
            for b in range(_K4 // 16):
                lanes = iota + b * 16
                cb = []
                for j in range(3):
                    jf = jnp.full((16,), j, jnp.int32)
                    a = (plsc.load_gather(alsr_v, [lanes, jf])
                         + plsc.load_gather(aldr_v, [lanes, jf]))
                    a = jnp.where(a > 0, a, 0.2 * a)
                    cb.append(jnp.exp(a)
                              * plsc.load_gather(rinr_v, [lanes, jf]))
                coef.append(cb)
            for e in range(_K4):
                b, lane = e // 16, e % 16
                cf = [_bc(coef[b][j], lane) for j in range(3)]
                for ch in range(C // 16):
                    dsl = pl.ds(ch * 16, 16)
                    msg_v[e, dsl] = (cf[0] * rows_v[e, 0, dsl]
                                     + cf[1] * rows_v[e, 1, dsl]
                                     + cf[2] * rows_v[e, 2, dsl])
            pltpu.sync_copy(msg_v, acc_sh.at[didx_v], add=True)
            return carry
        lax.fori_loop(0, npt // _K4, body, 0)
        plsc.subcore_barrier()
        pltpu.sync_copy(acc_sh.at[pl.ds(s * _RPT, _RPT)],
                        out_hbm.at[c, pl.ds(s * _RPT, _RPT)])
    return k


def _t1_body(x_ref, W1, b1, W2, b2, W3, b3, h_ref, st_ref):
    x = x_ref[...]
    h = jax.nn.softplus(_mm(x, W1[...]) + b1[...])
    h = jax.nn.softplus(_mm(h, W2[...]) + b2[...])
    h = _mm(h, W3[...]) + b3[...]
    h_ref[...] = h

    @pl.when(pl.program_id(0) == 0)
    def _():
        st_ref[...] = jnp.zeros_like(st_ref)

    st = jnp.concatenate(
        [jnp.sum(h, 0, keepdims=True), jnp.sum(h * h, 0, keepdims=True),
         jnp.zeros((6, h.shape[1]), jnp.float32)], axis=0)
    st_ref[...] += st


def _t1(x, p):
    return pl.pallas_call(
        _t1_body,
        grid=(_GRID,),
        in_specs=[_row_spec(_INDIM), _full_spec((_INDIM, 2 * _INDIM)),
                  _full_spec((1, 2 * _INDIM)), _full_spec((2 * _INDIM, _D4)),
                  _full_spec((1, _D4)), _full_spec((_D4, _D8)),
                  _full_spec((1, _D8))],
        out_specs=[_row_spec(_D8), _full_spec((8, _D8))],
        out_shape=[jax.ShapeDtypeStruct((_N, _D8), jnp.float32),
                   jax.ShapeDtypeStruct((8, _D8), jnp.float32)],
    )(x, p['nn1_W1'], p['nn1_b1'][None], p['nn1_W2'], p['nn1_b2'][None],
      p['nn1_W3'], p['nn1_b3'][None])


def _t2_body(h_ref, st_ref, degp_ref, g_ref, b_ref, W_ref,
             x0_ref, hwd_ref, dinv_ref):
    st = st_ref[...]
    mean = st[0:1, :] / _N
    var = st[1:2, :] / _N - mean * mean
    h = (h_ref[...] - mean) / jnp.sqrt(var + 1e-5) * g_ref[...] + b_ref[...]
    x0 = _leaky(h, 0.01)
    x0_ref[...] = x0
    deg = jnp.sum(degp_ref[...], axis=1)[:, None]
    dinv = jnp.where(deg > 0, lax.rsqrt(deg), 0.0)
    dinv_ref[...] = dinv
    hwd_ref[...] = _mm(x0, W_ref[...]) * dinv


def _t2(h, st, degp, p):
    return pl.pallas_call(
        _t2_body,
        grid=(_GRID,),
        in_specs=[_row_spec(_D8), _full_spec((8, _D8)),
                  _row_spec(32),
                  _full_spec((1, _D8)), _full_spec((1, _D8)),
                  _full_spec((_D8, _D4))],
        out_specs=[_row_spec(_D8), _row_spec(_D4), _row_spec(1)],
        out_shape=[jax.ShapeDtypeStruct((_N, _D8), jnp.float32),
                   jax.ShapeDtypeStruct((_N, _D4), jnp.float32),
                   jax.ShapeDtypeStruct((_N, 1), jnp.float32)],
    )(h, st, degp, p['bn0_g'][None], p['bn0_b'][None], p['gcn1_W'])


def _tgat_prep(a0, a1, dinv, b, W, a_s, a_d, din, cout, halves):
    def body(*refs):
        a0_ref, a1_ref, dinv_ref, b_ref, W_ref, as_ref, ad_ref = refs[:7]
        outs = refs[7:]
        xg = (a0_ref[...] + a1_ref[...]) * dinv_ref[...] + b_ref[...]
        hw = _mm(xg, W_ref[...])
        oi = 0
        for (off, wdt) in halves:
            for grp in range(2):
                cols = []
                for jj in range(3):
                    hh = 3 * grp + jj
                    if hh < _H:
                        cols.append(
                            hw[:, hh * cout + off:hh * cout + off + wdt])
                    else:
                        cols.append(jnp.zeros((_RB, wdt), jnp.float32))
                outs[oi][...] = jnp.concatenate(cols, axis=1)
                oi += 1
        z = jnp.zeros((_RB, 1), jnp.float32)
        for av in (as_ref[...], ad_ref[...]):
            cols = []
            for hh in range(_H):
                hwh = hw[:, hh * cout:(hh + 1) * cout]
                cols.append(jnp.sum(hwh * av[hh:hh + 1, :], axis=1,
                                    keepdims=True))
            cols.append(z)
            outs[oi][...] = jnp.concatenate(cols[0:3] + [z], axis=1)
            outs[oi + 1][...] = jnp.concatenate(cols[3:6] + [z], axis=1)
            oi += 2

    wdts = [wdt for (off, wdt) in halves for _ in range(2)]
    out_specs = ([_row_spec(3 * w) for w in wdts]
                 + [_row_spec(4)] * 4)
    out_shape = ([jax.ShapeDtypeStruct((_N, 3 * w), jnp.float32)
                  for w in wdts]
                 + [jax.ShapeDtypeStruct((_N, 4), jnp.float32)] * 4)
    return pl.pallas_call(
        body,
        grid=(_GRID,),
        in_specs=[_row_spec(din), _row_spec(din), _row_spec(1),
                  _full_spec((1, din)), _full_spec((din, _H * cout)),
                  _full_spec((_H, cout)), _full_spec((_H, cout))],
        out_specs=out_specs,
        out_shape=out_shape,
    )(a0, a1, dinv, b, W, a_s, a_d)


def _tden_body(dp_ref, rin_ref):
    den = jnp.sum(dp_ref[...], axis=0, keepdims=True)
    rin = 1.0 / (den + 1e-16)
    rin_ref[...] = jnp.concatenate(
        [rin, jnp.zeros((7, rin.shape[1]), jnp.float32)], axis=0)


def _tden(dp):
    flat = dp.reshape(16, 2 * 4 * _NPAD)
    out = pl.pallas_call(
        _tden_body,
        grid=(1,),
        in_specs=[_full_spec((16, 2 * 4 * _NPAD))],
        out_specs=_full_spec((8, 2 * 4 * _NPAD)),
        out_shape=jax.ShapeDtypeStruct((8, 2 * 4 * _NPAD), jnp.float32),
    )(flat)
    return out[0].reshape(2, 4, _NPAD)


def _t4a(m_parts, b, c):
    n = len(m_parts)
    ch = c // (n // 2)

    def body(*refs):
        m_refs = refs[:n]
        b_ref, x1p_ref, st_ref = refs[n], refs[n + 1], refs[n + 2]
        half = []
        for i in range(0, n, 2):
            half.append(m_refs[i][...] + m_refs[i + 1][...])
        x1p = jnp.concatenate(half, axis=1) * (1.0 / _H) + b_ref[...]
        x1p_ref[...] = x1p

        @pl.when(pl.program_id(0) == 0)
        def _():
            st_ref[...] = jnp.zeros_like(st_ref)

        st = jnp.concatenate(
            [jnp.sum(x1p, 0, keepdims=True),
             jnp.sum(x1p * x1p, 0, keepdims=True),
             jnp.zeros((6, x1p.shape[1]), jnp.float32)], axis=0)
        st_ref[...] += st

    return pl.pallas_call(
        body,
        grid=(_GRID,),
        in_specs=[_row_spec(ch)] * n + [_full_spec((1, c))],
        out_specs=[_row_spec(c), _full_spec((8, c))],
        out_shape=[jax.ShapeDtypeStruct((_N, c), jnp.float32),
                   jax.ShapeDtypeStruct((8, c), jnp.float32)],
    )(*m_parts, b)


def _t4b_body(x1p_ref, st_ref, g_ref, b_ref, x0_ref, Wa_ref, Wb_ref,
              dinv_ref, x1_ref, hwd_ref):
    st = st_ref[...]
    mean = st[0:1, :] / _N
    var = st[1:2, :] / _N - mean * mean
    h = (x1p_ref[...] - mean) / jnp.sqrt(var + 1e-5) * g_ref[...] + b_ref[...]
    x1 = _leaky(h, 0.01)
    x1_ref[...] = x1
    hwd_ref[...] = (_mm(x0_ref[...], Wa_ref[...])
                    + _mm(x1, Wb_ref[...])) * dinv_ref[...]


def _t4b(x1p, st, x0, dinv, p):
    W = p['gcn2_W']
    return pl.pallas_call(
        _t4b_body,
        grid=(_GRID,),
        in_specs=[_row_spec(_D4), _full_spec((8, _D4)), _full_spec((1, _D4)),
                  _full_spec((1, _D4)), _row_spec(_D8),
                  _full_spec((_D8, _D2)), _full_spec((_D4, _D2)),
                  _row_spec(1)],
        out_specs=[_row_spec(_D4), _row_spec(_D2)],
        out_shape=[jax.ShapeDtypeStruct((_N, _D4), jnp.float32),
                   jax.ShapeDtypeStruct((_N, _D2), jnp.float32)],
    )(x1p, st, p['bn1_g'][None], p['bn1_b'][None], x0,
      W[:_D8], W[_D8:], dinv)


def _t6_body(x2p_ref, x0_ref, x1_ref, W1a, W1b, W1c, b1, W2, b2, W3, b3,
             W4, b4, x2_ref, pr_ref):
    x2 = _leaky(x2p_ref[...], 0.01)
    x2_ref[...] = x2
    h = (_mm(x0_ref[...], W1a[...]) + _mm(x1_ref[...], W1b[...])
         + _mm(x2, W1c[...]) + b1[...])
    h = jax.nn.softplus(h)
    h = jax.nn.softplus(_mm(h, W2[...]) + b2[...])
    h = jax.nn.softplus(_mm(h, W3[...]) + b3[...])
    pr_ref[...] = jax.nn.sigmoid(_mm(h, W4[...]) + b4[...])


def _t6(x2p, x0, x1, p):
    W1 = p['pr_W1']
    c2, c4, c8 = _CAT // 2, _CAT // 4, _CAT // 8
    return pl.pallas_call(
        _t6_body,
        grid=(_GRID,),
        in_specs=[_row_spec(_D78), _row_spec(_D8), _row_spec(_D4),
                  _full_spec((_D8, c2)), _full_spec((_D4, c2)),
                  _full_spec((_D78, c2)), _full_spec((1, c2)),
                  _full_spec((c2, c4)), _full_spec((1, c4)),
                  _full_spec((c4, c8)), _full_spec((1, c8)),
                  _full_spec((c8, 1)), _full_spec((1, 1))],
        out_specs=[_row_spec(_D78), _row_spec(1)],
        out_shape=[jax.ShapeDtypeStruct((_N, _D78), jnp.float32),
                   jax.ShapeDtypeStruct((_N, 1), jnp.float32)],
    )(x2p, x0, x1, W1[:_D8], W1[_D8:_D8 + _D4], W1[_D8 + _D4:],
      p['pr_b1'][None], p['pr_W2'], p['pr_b2'][None], p['pr_W3'],
      p['pr_b3'][None], p['pr_W4'], p['pr_b4'][None])


def _padn(a):
    return jnp.pad(a, ((0, _NPAD - _N), (0, 0)))


def _pad4(a):
    return jnp.pad(a, ((0, _NPAD - _N), (0, 12)))


def _pad16r(a):
    return jnp.pad(a.T, ((0, 0), (0, 12)))


def _padr(a, w):
    return _padn(a).reshape(_NPAD, 3, w)


def kernel(x, edge_index, params):
    p = params
    loop = jnp.arange(_N, dtype=edge_index.dtype)
    padi = jnp.full((_EP - _E - _N,), _N, edge_index.dtype)
    src = jnp.concatenate([edge_index[0], loop, padi])
    dst = jnp.concatenate([edge_index[1], loop, padi])

    h, st0 = _t1(x, p)
    degp = _make_deg()(dst)
    x0, hwd1, dinv = _t2(h, st0, degp[:, :_N].T, p)

    a1p = _make_gcn(_D4)(_padn(hwd1), src, dst)
    hw1a, hw1b, als1a, als1b, ald1a, ald1b = _tgat_prep(
        a1p[0, :_N], a1p[1, :_N], dinv, p['gcn1_b'][None], p['gat1_W'],
        p['gat1_as'], p['gat1_ad'], _D4, _D4, [(0, _D4)])
    als1a, als1b = _pad4(als1a), _pad4(als1b)
    ald1a, ald1b = _pad4(ald1a), _pad4(ald1b)
    den1p = _make_den()(als1a, als1b, ald1a, ald1b, src, dst)
    rin1 = _tden(den1p)
    m1 = _make_gat(_D4)(_padr(hw1a, _D4), _padr(hw1b, _D4), als1a, als1b,
                        ald1a, ald1b, _pad16r(rin1[0]), _pad16r(rin1[1]),
                        src, dst)
    x1p, st1 = _t4a([m1[0, :_N], m1[1, :_N]], p['gat1_b'][None], _D4)
    x1, hwd2 = _t4b(x1p, st1, x0, dinv, p)

    a2p = _make_gcn(_D2)(_padn(hwd2), src, dst)
    hv = _D78 // 2
    hw2a1, hw2b1, hw2a2, hw2b2, als2a, als2b, ald2a, ald2b = _tgat_prep(
        a2p[0, :_N], a2p[1, :_N], dinv, p['gcn2_b'][None], p['gat2_W'],
        p['gat2_as'], p['gat2_ad'], _D2, _D78, [(0, hv), (hv, hv)])
    als2a, als2b = _pad4(als2a), _pad4(als2b)
    ald2a, ald2b = _pad4(ald2a), _pad4(ald2b)
    den2p = _make_den()(als2a, als2b, ald2a, ald2b, src, dst)
    rin2 = _tden(den2p)
    rin2a, rin2b = _pad16r(rin2[0]), _pad16r(rin2[1])
    m2h1 = _make_gat(hv)(_padr(hw2a1, hv), _padr(hw2b1, hv), als2a, als2b,
                         ald2a, ald2b, rin2a, rin2b, src, dst)
    m2h2 = _make_gat(hv)(_padr(hw2a2, hv), _padr(hw2b2, hv), als2a, als2b,
                         ald2a, ald2b, rin2a, rin2b, src, dst)
    x2p, _ = _t4a([m2h1[0, :_N], m2h1[1, :_N], m2h2[0, :_N], m2h2[1, :_N]],
                  p['gat2_b'][None], _D78)

    x2, probs = _t6(x2p, x0, x1, p)
    xf = jnp.concatenate([x0, x1, x2], axis=1)
    return xf, probs


# speedup vs baseline: 9.4358x; 1.1437x over previous
import functools
import jax
import jax.numpy as jnp
import numpy as np
from jax import lax
from jax.experimental import pallas as pl
from jax.experimental.pallas import tpu as pltpu
from jax.experimental.pallas import tpu_sc as plsc

_N = 10000
_E = 320000
_INDIM = 128
_OUT = 256
_H = 5
_D8 = _OUT // 8
_D4 = _OUT // 4
_D2 = _OUT // 2
_D78 = 7 * _OUT // 8
_CAT = _OUT + _D4

_RB = 400
_GRID = _N // _RB

_NPAD = 10112
_EP = 331776
_K2 = 128
_K4 = 64
_RPT = _NPAD // 16

_scparams = pltpu.CompilerParams(needs_layout_passes=False,
                                use_tc_tiling_on_sc=False)


@functools.lru_cache(maxsize=None)
def _scmesh():
    return plsc.VectorSubcoreMesh(core_axis_name="c", subcore_axis_name="s")


def _mm(a, b):
    return jnp.matmul(a, b)


def _row_spec(c):
    return pl.BlockSpec((_RB, c), lambda i: (i, 0))


def _full_spec(shape):
    return pl.BlockSpec(shape, lambda i: tuple(0 for _ in shape))


def _leaky(x, s):
    return jnp.where(x > 0, x, s * x)


def _zero16(ref, n):
    def body(i, carry):
        ref[pl.ds(i * 16, 16)] = jnp.zeros((16,), jnp.float32)
        return carry
    lax.fori_loop(0, n // 16, body, 0)


def _zero2d(ref, rows, cols):
    def body(i, carry):
        for ch in range(cols // 16):
            ref[i, pl.ds(ch * 16, 16)] = jnp.zeros((16,), jnp.float32)
        return carry
    lax.fori_loop(0, rows, body, 0)


def _bc(v, lane):
    idx = jnp.full((16, 1), lane, jnp.int32)
    dn = lax.GatherDimensionNumbers(offset_dims=(), collapsed_slice_dims=(0,),
                                    start_index_map=(0,))
    return lax.gather(v, idx, dn, (1,),
                      mode=lax.GatherScatterMode.PROMISE_IN_BOUNDS)


@functools.lru_cache(maxsize=None)
def _make_deg():
    @functools.partial(
        pl.kernel, mesh=_scmesh(),
        out_type=jax.ShapeDtypeStruct((32, _NPAD), jnp.float32),
        scratch_types=[pltpu.VMEM((_NPAD,), jnp.float32),
                       pltpu.VMEM((_K2,), jnp.int32)],
        compiler_params=_scparams)
    def k(dst_hbm, out_hbm, tab_v, idx_v):
        c = lax.axis_index("c")
        s = lax.axis_index("s")
        w = s * 2 + c
        _zero16(tab_v, _NPAD)
        ones = jnp.full((16,), 1.0, jnp.float32)
        npt = _EP // 32
        base0 = w * npt

        def body(wi, carry):
            pltpu.sync_copy(dst_hbm.at[pl.ds(base0 + wi * _K2, _K2)], idx_v)
            for b in range(_K2 // 16):
                plsc.addupdate_scatter(tab_v, [idx_v[pl.ds(b * 16, 16)]],
                                       ones)
            return carry
        lax.fori_loop(0, npt // _K2, body, 0)
        pltpu.sync_copy(tab_v, out_hbm.at[w])
    return k


@functools.lru_cache(maxsize=None)
def _make_gcn(C):
    @functools.partial(
        pl.kernel, mesh=_scmesh(),
        out_type=jax.ShapeDtypeStruct((2, _NPAD, C), jnp.float32),
        scratch_types=[pltpu.VMEM((_K2,), jnp.int32),
                       pltpu.VMEM((_K2,), jnp.int32),
                       pltpu.VMEM((_K2, C), jnp.float32),
                       pltpu.VMEM((8, C), jnp.float32),
                       pltpu.VMEM_SHARED((_NPAD, C), jnp.float32),
                       pltpu.SemaphoreType.DMA],
        compiler_params=_scparams)
    def k(tab_hbm, src_hbm, dst_hbm, out_hbm,
          sidx_v, didx_v, rows_v, zbuf_v, acc_sh, sem):
        c = lax.axis_index("c")
        s = lax.axis_index("s")
        w = s * 2 + c
        _zero2d(zbuf_v, 8, C)

        def zbody(i, carry):
            pltpu.sync_copy(zbuf_v, acc_sh.at[pl.ds(s * _RPT + i * 8, 8)])
            return carry
        lax.fori_loop(0, _RPT // 8, zbody, 0)
        plsc.subcore_barrier()

        npt = _EP // 32
        base0 = w * npt

        def body(wi, carry):
            base = base0 + wi * _K2
            pltpu.sync_copy(src_hbm.at[pl.ds(base, _K2)], sidx_v)
            pltpu.async_copy(tab_hbm.at[sidx_v], rows_v, sem).wait()
            pltpu.sync_copy(dst_hbm.at[pl.ds(base, _K2)], didx_v)
            pltpu.sync_copy(rows_v, acc_sh.at[didx_v], add=True)
            return carry
        lax.fori_loop(0, npt // _K2, body, 0)
        plsc.subcore_barrier()
        pltpu.sync_copy(acc_sh.at[pl.ds(s * _RPT, _RPT)],
                        out_hbm.at[c, pl.ds(s * _RPT, _RPT)])
    return k


@functools.lru_cache(maxsize=None)
def _make_den():
    @functools.partial(
        pl.kernel, mesh=_scmesh(),
        out_type=jax.ShapeDtypeStruct((16, 2, 4, _NPAD), jnp.float32),
        scratch_types=[pltpu.VMEM((4, _NPAD), jnp.float32),
                       pltpu.VMEM((64,), jnp.int32),
                       pltpu.VMEM((64,), jnp.int32),
                       pltpu.VMEM((64, 16), jnp.float32),
                       pltpu.VMEM((64, 16), jnp.float32),
                       pltpu.SemaphoreType.DMA,
                       pltpu.SemaphoreType.DMA],
        compiler_params=_scparams)
    def k(alsa_hbm, alsb_hbm, alda_hbm, aldb_hbm, src_hbm, dst_hbm, out_hbm,
          den_v, sidx_v, didx_v, alsr_v, aldr_v, sem, sem2):
        c = lax.axis_index("c")
        s = lax.axis_index("s")
        for j in range(4):
            _zero16(den_v.at[j], _NPAD)

        npt = _EP // 16
        base0 = s * npt
        iota = lax.iota(jnp.int32, 16)

        def body(wi, carry):
            base = base0 + wi * 64
            pltpu.sync_copy(src_hbm.at[pl.ds(base, 64)], sidx_v)
            pltpu.sync_copy(dst_hbm.at[pl.ds(base, 64)], didx_v)

            @pl.when(c == 0)
            def _():
                pltpu.async_copy(alsa_hbm.at[sidx_v], alsr_v, sem).wait()
                pltpu.async_copy(alda_hbm.at[didx_v], aldr_v, sem2).wait()

            @pl.when(c == 1)
            def _():
                pltpu.async_copy(alsb_hbm.at[sidx_v], alsr_v, sem).wait()
                pltpu.async_copy(aldb_hbm.at[didx_v], aldr_v, sem2).wait()

            for b in range(4):
                lanes = iota + b * 16
                didx = didx_v[pl.ds(b * 16, 16)]
                for j in range(3):
                    jf = jnp.full((16,), j, jnp.int32)
                    a = (plsc.load_gather(alsr_v, [lanes, jf])
                         + plsc.load_gather(aldr_v, [lanes, jf]))
                    a = jnp.where(a > 0, a, 0.2 * a)
                    plsc.addupdate_scatter(den_v, [jf, didx], jnp.exp(a))
            return carry
        lax.fori_loop(0, npt // 64, body, 0)
        pltpu.sync_copy(den_v, out_hbm.at[s, c])
    return k


@functools.lru_cache(maxsize=None)
def _make_gat(C):
    @functools.partial(
        pl.kernel, mesh=_scmesh(),
        out_type=jax.ShapeDtypeStruct((2, _NPAD, C), jnp.float32),
        scratch_types=[pltpu.VMEM((_K4,), jnp.int32),
                       pltpu.VMEM((_K4,), jnp.int32),
                       pltpu.VMEM((_K4, 16), jnp.float32),
                       pltpu.VMEM((_K4, 16), jnp.float32),
                       pltpu.VMEM((_K4, 16), jnp.float32),
                       pltpu.VMEM((_K4, 3, C), jnp.float32),
                       pltpu.VMEM((_K4, C), jnp.float32),
                       pltpu.VMEM((8, C), jnp.float32),
                       pltpu.VMEM_SHARED((_NPAD, C), jnp.float32),
                       pltpu.SemaphoreType.DMA,
                       pltpu.SemaphoreType.DMA,
                       pltpu.SemaphoreType.DMA,
                       pltpu.SemaphoreType.DMA],
        compiler_params=_scparams)
    def k(hwa_hbm, hwb_hbm, alsa_hbm, alsb_hbm, alda_hbm, aldb_hbm,
          rina_hbm, rinb_hbm, src_hbm, dst_hbm, out_hbm,
          sidx_v, didx_v, alsr_v, aldr_v, rinr_v, rows_v, msg_v, zbuf_v,
          acc_sh, sem, sem2, sem3, sem4):
        c = lax.axis_index("c")
        s = lax.axis_index("s")
        _zero2d(zbuf_v, 8, C)

        def zbody(i, carry):
            pltpu.sync_copy(zbuf_v, acc_sh.at[pl.ds(s * _RPT + i * 8, 8)])
            return carry
        lax.fori_loop(0, _RPT // 8, zbody, 0)
        plsc.subcore_barrier()

        npt = _EP // 16
        base0 = s * npt
        iota = lax.iota(jnp.int32, 16)

        def body(wi, carry):
            base = base0 + wi * _K4
            pltpu.sync_copy(src_hbm.at[pl.ds(base, _K4)], sidx_v)
            pltpu.sync_copy(dst_hbm.at[pl.ds(base, _K4)], didx_v)

            @pl.when(c == 0)
            def _():
                pltpu.async_copy(hwa_hbm.at[sidx_v], rows_v, sem).wait()
                pltpu.async_copy(alsa_hbm.at[sidx_v], alsr_v, sem2).wait()
                pltpu.async_copy(alda_hbm.at[didx_v], aldr_v, sem3).wait()
                pltpu.async_copy(rina_hbm.at[didx_v], rinr_v, sem4).wait()

            @pl.when(c == 1)
            def _():
                pltpu.async_copy(hwb_hbm.at[sidx_v], rows_v, sem).wait()
                pltpu.async_copy(alsb_hbm.at[sidx_v], alsr_v, sem2).wait()
                pltpu.async_copy(aldb_hbm.at[didx_v], aldr_v, sem3).wait()
                pltpu.async_copy(rinb_hbm.at[didx_v], rinr_v, sem4).wait()

            coef = []
            for b in range(_K4 // 16):
                lanes = iota + b * 16
                cb = []
                for j in range(3):
                    jf = jnp.full((16,), j, jnp.int32)
                    a = (plsc.load_gather(alsr_v, [lanes, jf])
                         + plsc.load_gather(aldr_v, [lanes, jf]))
                    a = jnp.where(a > 0, a, 0.2 * a)
                    cb.append(jnp.exp(a)
                              * plsc.load_gather(rinr_v, [lanes, jf]))
                coef.append(cb)
            for e in range(_K4):
                b, lane = e // 16, e % 16
                cf = [_bc(coef[b][j], lane) for j in range(3)]
                for ch in range(C // 16):
                    dsl = pl.ds(ch * 16, 16)
                    msg_v[e, dsl] = (cf[0] * rows_v[e, 0, dsl]
                                     + cf[1] * rows_v[e, 1, dsl]
                                     + cf[2] * rows_v[e, 2, dsl])
            pltpu.sync_copy(msg_v, acc_sh.at[didx_v], add=True)
            return carry
        lax.fori_loop(0, npt // _K4, body, 0)
        plsc.subcore_barrier()
        pltpu.sync_copy(acc_sh.at[pl.ds(s * _RPT, _RPT)],
                        out_hbm.at[c, pl.ds(s * _RPT, _RPT)])
    return k


def _t1_body(x_ref, W1, b1, W2, b2, W3, b3, h_ref, st_ref):
    x = x_ref[...]
    h = jax.nn.softplus(_mm(x, W1[...]) + b1[...])
    h = jax.nn.softplus(_mm(h, W2[...]) + b2[...])
    h = _mm(h, W3[...]) + b3[...]
    h_ref[...] = h

    @pl.when(pl.program_id(0) == 0)
    def _():
        st_ref[...] = jnp.zeros_like(st_ref)

    st = jnp.concatenate(
        [jnp.sum(h, 0, keepdims=True), jnp.sum(h * h, 0, keepdims=True),
         jnp.zeros((6, h.shape[1]), jnp.float32)], axis=0)
    st_ref[...] += st


def _t1(x, p):
    return pl.pallas_call(
        _t1_body,
        grid=(_GRID,),
        in_specs=[_row_spec(_INDIM), _full_spec((_INDIM, 2 * _INDIM)),
                  _full_spec((1, 2 * _INDIM)), _full_spec((2 * _INDIM, _D4)),
                  _full_spec((1, _D4)), _full_spec((_D4, _D8)),
                  _full_spec((1, _D8))],
        out_specs=[_row_spec(_D8), _full_spec((8, _D8))],
        out_shape=[jax.ShapeDtypeStruct((_N, _D8), jnp.float32),
                   jax.ShapeDtypeStruct((8, _D8), jnp.float32)],
    )(x, p['nn1_W1'], p['nn1_b1'][None], p['nn1_W2'], p['nn1_b2'][None],
      p['nn1_W3'], p['nn1_b3'][None])


def _t2_body(h_ref, st_ref, degp_ref, g_ref, b_ref, W_ref,
             x0_ref, hwd_ref, dinv_ref):
    st = st_ref[...]
    mean = st[0:1, :] / _N
    var = st[1:2, :] / _N - mean * mean
    h = (h_ref[...] - mean) / jnp.sqrt(var + 1e-5) * g_ref[...] + b_ref[...]
    x0 = _leaky(h, 0.01)
    x0_ref[...] = x0
    deg = jnp.sum(degp_ref[...], axis=1)[:, None]
    dinv = jnp.where(deg > 0, lax.rsqrt(deg), 0.0)
    dinv_ref[...] = dinv
    hwd_ref[...] = _mm(x0, W_ref[...]) * dinv


def _t2(h, st, degp, p):
    return pl.pallas_call(
        _t2_body,
        grid=(_GRID,),
        in_specs=[_row_spec(_D8), _full_spec((8, _D8)),
                  _row_spec(32),
                  _full_spec((1, _D8)), _full_spec((1, _D8)),
                  _full_spec((_D8, _D4))],
        out_specs=[_row_spec(_D8), _row_spec(_D4), _row_spec(1)],
        out_shape=[jax.ShapeDtypeStruct((_N, _D8), jnp.float32),
                   jax.ShapeDtypeStruct((_N, _D4), jnp.float32),
                   jax.ShapeDtypeStruct((_N, 1), jnp.float32)],
    )(h, st, degp, p['bn0_g'][None], p['bn0_b'][None], p['gcn1_W'])


def _tgat_prep(a0, a1, dinv, b, W, a_s, a_d, din, cout, halves):
    def body(*refs):
        a0_ref, a1_ref, dinv_ref, b_ref, W_ref, as_ref, ad_ref = refs[:7]
        outs = refs[7:]
        xg = (a0_ref[...] + a1_ref[...]) * dinv_ref[...] + b_ref[...]
        hw = _mm(xg, W_ref[...])
        oi = 0
        for (off, wdt) in halves:
            for grp in range(2):
                cols = []
                for jj in range(3):
                    hh = 3 * grp + jj
                    if hh < _H:
                        cols.append(
                            hw[:, hh * cout + off:hh * cout + off + wdt])
                    else:
                        cols.append(jnp.zeros((_RB, wdt), jnp.float32))
                outs[oi][...] = jnp.concatenate(cols, axis=1)
                oi += 1
        z = jnp.zeros((_RB, 1), jnp.float32)
        for av in (as_ref[...], ad_ref[...]):
            cols = []
            for hh in range(_H):
                hwh = hw[:, hh * cout:(hh + 1) * cout]
                cols.append(jnp.sum(hwh * av[hh:hh + 1, :], axis=1,
                                    keepdims=True))
            cols.append(z)
            outs[oi][...] = jnp.concatenate(cols[0:3] + [z], axis=1)
            outs[oi + 1][...] = jnp.concatenate(cols[3:6] + [z], axis=1)
            oi += 2

    wdts = [wdt for (off, wdt) in halves for _ in range(2)]
    out_specs = ([_row_spec(3 * w) for w in wdts]
                 + [_row_spec(4)] * 4)
    out_shape = ([jax.ShapeDtypeStruct((_N, 3 * w), jnp.float32)
                  for w in wdts]
                 + [jax.ShapeDtypeStruct((_N, 4), jnp.float32)] * 4)
    return pl.pallas_call(
        body,
        grid=(_GRID,),
        in_specs=[_row_spec(din), _row_spec(din), _row_spec(1),
                  _full_spec((1, din)), _full_spec((din, _H * cout)),
                  _full_spec((_H, cout)), _full_spec((_H, cout))],
        out_specs=out_specs,
        out_shape=out_shape,
    )(a0, a1, dinv, b, W, a_s, a_d)


def _tden_body(dp_ref, rin_ref):
    den = jnp.sum(dp_ref[...], axis=0, keepdims=True)
    rin = 1.0 / (den + 1e-16)
    rin_ref[...] = jnp.concatenate(
        [rin, jnp.zeros((7, rin.shape[1]), jnp.float32)], axis=0)


def _tden(dp):
    flat = dp.reshape(16, 2 * 4 * _NPAD)
    out = pl.pallas_call(
        _tden_body,
        grid=(1,),
        in_specs=[_full_spec((16, 2 * 4 * _NPAD))],
        out_specs=_full_spec((8, 2 * 4 * _NPAD)),
        out_shape=jax.ShapeDtypeStruct((8, 2 * 4 * _NPAD), jnp.float32),
    )(flat)
    return out[0].reshape(2, 4, _NPAD)


def _t4a(m_parts, b, c):
    n = len(m_parts)
    ch = c // (n // 2)

    def body(*refs):
        m_refs = refs[:n]
        b_ref, x1p_ref, st_ref = refs[n], refs[n + 1], refs[n + 2]
        half = []
        for i in range(0, n, 2):
            half.append(m_refs[i][...] + m_refs[i + 1][...])
        x1p = jnp.concatenate(half, axis=1) * (1.0 / _H) + b_ref[...]
        x1p_ref[...] = x1p

        @pl.when(pl.program_id(0) == 0)
        def _():
            st_ref[...] = jnp.zeros_like(st_ref)

        st = jnp.concatenate(
            [jnp.sum(x1p, 0, keepdims=True),
             jnp.sum(x1p * x1p, 0, keepdims=True),
             jnp.zeros((6, x1p.shape[1]), jnp.float32)], axis=0)
        st_ref[...] += st

    return pl.pallas_call(
        body,
        grid=(_GRID,),
        in_specs=[_row_spec(ch)] * n + [_full_spec((1, c))],
        out_specs=[_row_spec(c), _full_spec((8, c))],
        out_shape=[jax.ShapeDtypeStruct((_N, c), jnp.float32),
                   jax.ShapeDtypeStruct((8, c), jnp.float32)],
    )(*m_parts, b)


def _t4b_body(x1p_ref, st_ref, g_ref, b_ref, x0_ref, Wa_ref, Wb_ref,
              dinv_ref, x1_ref, hwd_ref):
    st = st_ref[...]
    mean = st[0:1, :] / _N
    var = st[1:2, :] / _N - mean * mean
    h = (x1p_ref[...] - mean) / jnp.sqrt(var + 1e-5) * g_ref[...] + b_ref[...]
    x1 = _leaky(h, 0.01)
    x1_ref[...] = x1
    hwd_ref[...] = (_mm(x0_ref[...], Wa_ref[...])
                    + _mm(x1, Wb_ref[...])) * dinv_ref[...]


def _t4b(x1p, st, x0, dinv, p):
    W = p['gcn2_W']
    return pl.pallas_call(
        _t4b_body,
        grid=(_GRID,),
        in_specs=[_row_spec(_D4), _full_spec((8, _D4)), _full_spec((1, _D4)),
                  _full_spec((1, _D4)), _row_spec(_D8),
                  _full_spec((_D8, _D2)), _full_spec((_D4, _D2)),
                  _row_spec(1)],
        out_specs=[_row_spec(_D4), _row_spec(_D2)],
        out_shape=[jax.ShapeDtypeStruct((_N, _D4), jnp.float32),
                   jax.ShapeDtypeStruct((_N, _D2), jnp.float32)],
    )(x1p, st, p['bn1_g'][None], p['bn1_b'][None], x0,
      W[:_D8], W[_D8:], dinv)


def _t6_body(x2p_ref, x0_ref, x1_ref, W1a, W1b, W1c, b1, W2, b2, W3, b3,
             W4, b4, x2_ref, pr_ref):
    x2 = _leaky(x2p_ref[...], 0.01)
    x2_ref[...] = x2
    h = (_mm(x0_ref[...], W1a[...]) + _mm(x1_ref[...], W1b[...])
         + _mm(x2, W1c[...]) + b1[...])
    h = jax.nn.softplus(h)
    h = jax.nn.softplus(_mm(h, W2[...]) + b2[...])
    h = jax.nn.softplus(_mm(h, W3[...]) + b3[...])
    pr_ref[...] = jax.nn.sigmoid(_mm(h, W4[...]) + b4[...])


def _t6(x2p, x0, x1, p):
    W1 = p['pr_W1']
    c2, c4, c8 = _CAT // 2, _CAT // 4, _CAT // 8
    return pl.pallas_call(
        _t6_body,
        grid=(_GRID,),
        in_specs=[_row_spec(_D78), _row_spec(_D8), _row_spec(_D4),
                  _full_spec((_D8, c2)), _full_spec((_D4, c2)),
                  _full_spec((_D78, c2)), _full_spec((1, c2)),
                  _full_spec((c2, c4)), _full_spec((1, c4)),
                  _full_spec((c4, c8)), _full_spec((1, c8)),
                  _full_spec((c8, 1)), _full_spec((1, 1))],
        out_specs=[_row_spec(_D78), _row_spec(1)],
        out_shape=[jax.ShapeDtypeStruct((_N, _D78), jnp.float32),
                   jax.ShapeDtypeStruct((_N, 1), jnp.float32)],
    )(x2p, x0, x1, W1[:_D8], W1[_D8:_D8 + _D4], W1[_D8 + _D4:],
      p['pr_b1'][None], p['pr_W2'], p['pr_b2'][None], p['pr_W3'],
      p['pr_b3'][None], p['pr_W4'], p['pr_b4'][None])


def _padn(a):
    return jnp.pad(a, ((0, _NPAD - _N), (0, 0)))


def _pad4(a):
    return jnp.pad(a, ((0, _NPAD - _N), (0, 12)))


def _pad16r(a):
    return jnp.pad(a.T, ((0, 0), (0, 12)))


def _padr(a, w):
    return _padn(a).reshape(_NPAD, 3, w)


def kernel(x, edge_index, params):
    p = params
    loop = jnp.arange(_N, dtype=edge_index.dtype)
    padi = jnp.full((_EP - _E - _N,), _N, edge_index.dtype)
    src = jnp.concatenate([edge_index[0], loop, padi])
    dst = jnp.concatenate([edge_index[1], loop, padi])

    h, st0 = _t1(x, p)
    degp = _make_deg()(dst)
    x0, hwd1, dinv = _t2(h, st0, degp[:, :_N].T, p)

    a1p = _make_gcn(_D4)(_padn(hwd1), src, dst)
    hw1a, hw1b, als1a, als1b, ald1a, ald1b = _tgat_prep(
        a1p[0, :_N], a1p[1, :_N], dinv, p['gcn1_b'][None], p['gat1_W'],
        p['gat1_as'], p['gat1_ad'], _D4, _D4, [(0, _D4)])
    als1a, als1b = _pad4(als1a), _pad4(als1b)
    ald1a, ald1b = _pad4(ald1a), _pad4(ald1b)
    den1p = _make_den()(als1a, als1b, ald1a, ald1b, src, dst)
    rin1 = _tden(den1p)
    m1 = _make_gat(_D4)(_padr(hw1a, _D4), _padr(hw1b, _D4), als1a, als1b,
                        ald1a, ald1b, _pad16r(rin1[0]), _pad16r(rin1[1]),
                        src, dst)
    x1p, st1 = _t4a([m1[0, :_N], m1[1, :_N]], p['gat1_b'][None], _D4)
    x1, hwd2 = _t4b(x1p, st1, x0, dinv, p)

    a2p = _make_gcn(_D2)(_padn(hwd2), src, dst)
    hv = _D78 // 2
    hw2a1, hw2b1, hw2a2, hw2b2, als2a, als2b, ald2a, ald2b = _tgat_prep(
        a2p[0, :_N], a2p[1, :_N], dinv, p['gcn2_b'][None], p['gat2_W'],
        p['gat2_as'], p['gat2_ad'], _D2, _D78, [(0, hv), (hv, hv)])
    als2a, als2b = _pad4(als2a), _pad4(als2b)
    ald2a, ald2b = _pad4(ald2a), _pad4(ald2b)
    den2p = _make_den()(als2a, als2b, ald2a, ald2b, src, dst)
    rin2 = _tden(den2p)
    rin2a, rin2b = _pad16r(rin2[0]), _pad16r(rin2[1])
    m2h1 = _make_gat(hv)(_padr(hw2a1, hv), _padr(hw2b1, hv), als2a, als2b,
                         ald2a, ald2b, rin2a, rin2b, src, dst)
    m2h2 = _make_gat(hv)(_padr(hw2a2, hv), _padr(hw2b2, hv), als2a, als2b,
                         ald2a, ald2b, rin2a, rin2b, src, dst)
    x2p, _ = _t4a([m2h1[0, :_N], m2h1[1, :_N], m2h2[0, :_N], m2h2[1, :_N]],
                  p['gat2_b'][None], _D78)

    x2, probs = _t6(x2p, x0, x1, p)
    xf = jnp.concatenate([x0, x1, x2], axis=1)
    return xf, probs
